# Initial kernel scaffold; baseline (speedup 1.0000x reference)
import jax
import jax.numpy as jnp
from jax.experimental import pallas as pl


def kernel(x, edge_index, edge_attr, We1, be1, We2, be2, Wn1, bn1, Wn2, bn2):
    raise NotImplementedError("write your pallas kernel here")


# speedup vs baseline: 2.4166x; 2.4166x over previous
import functools

import jax
import jax.numpy as jnp
from jax import lax
from jax.experimental import pallas as pl
from jax.experimental.pallas import tpu as pltpu
from jax.experimental.pallas import tpu_sc as plsc

N_NODES = 10000
N_EDGES = 320000
POS = 3
FE = 128
EA = 16
EH = 32
MD = 16
NH = 256

W = 128
EPAD = 323584
NWIN = EPAD // W
NPAD = 10016
_SUB = 16
_STRIPE = NPAD // _SUB

_HP = lax.Precision.HIGHEST


def _silu(v):
    return v * jax.nn.sigmoid(v)


def _pre_body(f_ref, wa_ref, wb_ref, a_ref, b_ref):
    f = f_ref[...]
    a_ref[...] = jnp.dot(f, wa_ref[...], precision=_HP)
    b_ref[...] = jnp.dot(f, wb_ref[...], precision=_HP)


def _pre(feats, wa, wb):
    R = 2000
    return pl.pallas_call(
        _pre_body,
        grid=(N_NODES // R,),
        in_specs=[
            pl.BlockSpec((R, FE), lambda i: (i, 0)),
            pl.BlockSpec((FE, EH), lambda i: (0, 0)),
            pl.BlockSpec((FE, EH), lambda i: (0, 0)),
        ],
        out_specs=[
            pl.BlockSpec((R, EH), lambda i: (i, 0)),
            pl.BlockSpec((R, EH), lambda i: (i, 0)),
        ],
        out_shape=[jax.ShapeDtypeStruct((N_NODES, EH), jnp.float32)] * 2,
    )(feats, wa, wb)


def _gather(a, b, dst2d, src2d):
    mesh = plsc.VectorSubcoreMesh(core_axis_name="core", subcore_axis_name="subcore")

    @functools.partial(
        pl.kernel,
        out_type=jax.ShapeDtypeStruct((EPAD, EH), jnp.float32),
        mesh=mesh,
        compiler_params=pltpu.CompilerParams(use_tc_tiling_on_sc=False),
        scratch_types=[
            pltpu.VMEM((W, EH), jnp.float32),
            pltpu.SemaphoreType.DMA,
            pltpu.SemaphoreType.DMA,
        ],
    )
    def kern(a_hbm, b_hbm, d_hbm, s_hbm, o_hbm, tmp, sem_a, sem_b):
        def body(d_vmem, s_vmem, o_vmem):
            cpa = pltpu.async_copy(a_hbm.at[d_vmem.at[0]], o_vmem, sem_a)
            cpb = pltpu.async_copy(b_hbm.at[s_vmem.at[0]], tmp, sem_b)
            cpa.wait()
            cpb.wait()

            @pl.loop(0, W)
            def _row(r):
                @pl.loop(0, EH, step=16)
                def _col(c):
                    slc = (pl.ds(r, 1), pl.ds(c, 16))
                    o_vmem.at[slc][...] = o_vmem.at[slc][...] + tmp.at[slc][...]

        pltpu.emit_pipeline(
            body,
            grid=(NWIN,),
            in_specs=[
                pl.BlockSpec((1, W), lambda i: (i, 0)),
                pl.BlockSpec((1, W), lambda i: (i, 0)),
            ],
            out_specs=[pl.BlockSpec((W, EH), lambda i: (i, 0))],
            core_axis_name=("core", "subcore"),
            dimension_semantics=(pltpu.PARALLEL,),
        )(d_hbm, s_hbm, o_hbm)

    return kern(a, b, dst2d, src2d)


def _edge_body(eg_ref, ea_ref, wc_ref, b1_ref, w2_ref, b2_ref, o_ref):
    z = eg_ref[...] + jnp.dot(ea_ref[...], wc_ref[...], precision=_HP) + b1_ref[...]
    m1 = _silu(z)
    y = jnp.dot(m1, w2_ref[...], precision=_HP) + b2_ref[...]
    o_ref[...] = _silu(y)


def _edge(eg, ea, wc, b1, w2, b2):
    Be = 4096
    return pl.pallas_call(
        _edge_body,
        grid=(EPAD // Be,),
        in_specs=[
            pl.BlockSpec((Be, EH), lambda i: (i, 0)),
            pl.BlockSpec((Be, EA), lambda i: (i, 0)),
            pl.BlockSpec((EA, EH), lambda i: (0, 0)),
            pl.BlockSpec((1, EH), lambda i: (0, 0)),
            pl.BlockSpec((EH, MD), lambda i: (0, 0)),
            pl.BlockSpec((1, MD), lambda i: (0, 0)),
        ],
        out_specs=pl.BlockSpec((Be, MD), lambda i: (i, 0)),
        out_shape=jax.ShapeDtypeStruct((EPAD, MD), jnp.float32),
    )(eg, ea, wc, b1, w2, b2)


def _scatter(m2, dst2d):
    mesh = plsc.VectorSubcoreMesh(core_axis_name="core", subcore_axis_name="subcore")

    @functools.partial(
        pl.kernel,
        out_type=jax.ShapeDtypeStruct((2, NPAD, MD), jnp.float32),
        mesh=mesh,
        compiler_params=pltpu.CompilerParams(use_tc_tiling_on_sc=False),
        scratch_types=[
            pltpu.VMEM((_STRIPE, MD), jnp.float32),
            pltpu.VMEM_SHARED((NPAD, MD), jnp.float32),
        ],
    )
    def kern(m_hbm, d_hbm, o_hbm, zbuf, acc):
        cid = lax.axis_index("core")
        sid = lax.axis_index("subcore")

        @pl.loop(0, _STRIPE)
        def _z(r):
            zbuf.at[pl.ds(r, 1), pl.ds(0, MD)][...] = jnp.zeros((1, MD), jnp.float32)

        pltpu.sync_copy(zbuf, acc.at[pl.ds(sid * _STRIPE, _STRIPE)])
        plsc.subcore_barrier()

        def body(m_vmem, d_vmem):
            pltpu.sync_copy(m_vmem, acc.at[d_vmem.at[0]], add=True)

        pltpu.emit_pipeline(
            body,
            grid=(NWIN,),
            in_specs=[
                pl.BlockSpec((W, MD), lambda i: (i, 0)),
                pl.BlockSpec((1, W), lambda i: (i, 0)),
            ],
            out_specs=[],
            core_axis_name=("core", "subcore"),
            dimension_semantics=(pltpu.PARALLEL,),
        )(m_hbm, d_hbm)

        plsc.subcore_barrier()
        pltpu.sync_copy(acc.at[pl.ds(sid * _STRIPE, _STRIPE)],
                        o_hbm.at[cid, pl.ds(sid * _STRIPE, _STRIPE)])

    return kern(m2, dst2d)


def _node_body(f_ref, p0_ref, p1_ref, wf_ref, wa_ref, b1_ref, w2_ref, b2_ref, o_ref):
    f = f_ref[...]
    agg = p0_ref[...] + p1_ref[...]
    h = (jnp.dot(f, wf_ref[...], precision=_HP)
         + jnp.dot(agg, wa_ref[...], precision=_HP)
         + b1_ref[...])
    h = _silu(h)
    o_ref[...] = f + jnp.dot(h, w2_ref[...], precision=_HP) + b2_ref[...]


def _node(feats, p0, p1, wf, wa, b1, w2, b2):
    R = 2000
    return pl.pallas_call(
        _node_body,
        grid=(N_NODES // R,),
        in_specs=[
            pl.BlockSpec((R, FE), lambda i: (i, 0)),
            pl.BlockSpec((R, MD), lambda i: (i, 0)),
            pl.BlockSpec((R, MD), lambda i: (i, 0)),
            pl.BlockSpec((FE, NH), lambda i: (0, 0)),
            pl.BlockSpec((MD, NH), lambda i: (0, 0)),
            pl.BlockSpec((1, NH), lambda i: (0, 0)),
            pl.BlockSpec((NH, FE), lambda i: (0, 0)),
            pl.BlockSpec((1, FE), lambda i: (0, 0)),
        ],
        out_specs=pl.BlockSpec((R, FE), lambda i: (i, 0)),
        out_shape=jax.ShapeDtypeStruct((N_NODES, FE), jnp.float32),
    )(feats, p0, p1, wf, wa, b1, w2, b2)


def kernel(x, edge_index, edge_attr, We1, be1, We2, be2, Wn1, bn1, Wn2, bn2):
    feats = x[:, POS:]
    src = edge_index[0]
    dst = edge_index[1]
    pad = EPAD - N_EDGES
    dst_g = jnp.pad(dst, (0, pad)).reshape(NWIN, W)
    src_g = jnp.pad(src, (0, pad)).reshape(NWIN, W)
    dst_s = jnp.pad(dst, (0, pad), constant_values=N_NODES).reshape(NWIN, W)
    ea_p = jnp.pad(edge_attr, ((0, pad), (0, 0)))

    a, b = _pre(feats, We1[:FE], We1[FE:2 * FE])
    eg = _gather(a, b, dst_g, src_g)
    m2 = _edge(eg, ea_p, We1[2 * FE:], be1.reshape(1, EH), We2,
               be2.reshape(1, MD))
    parts = _scatter(m2, dst_s)
    feats_out = _node(feats, parts[0], parts[1], Wn1[:FE], Wn1[FE:],
                      bn1.reshape(1, NH), Wn2, bn2.reshape(1, FE))
    return jnp.concatenate([x[:, :POS], feats_out], axis=1)


# speedup vs baseline: 2.6200x; 1.0842x over previous
import functools

import jax
import jax.numpy as jnp
from jax import lax
from jax.experimental import pallas as pl
from jax.experimental.pallas import tpu as pltpu
from jax.experimental.pallas import tpu_sc as plsc

N_NODES = 10000
N_EDGES = 320000
POS = 3
FE = 128
EA = 16
EH = 32
MD = 16
NH = 256

W = 125
NWIN = N_EDGES // W
_SUB = 16
_STRIPE = N_NODES // _SUB

_HP = lax.Precision.HIGHEST


def _silu(v):
    return v * jax.nn.sigmoid(v)


def _pre_body(f_ref, wa_ref, wb_ref, a_ref, b_ref):
    f = f_ref[...]
    a_ref[...] = jnp.dot(f, wa_ref[...], precision=_HP)
    b_ref[...] = jnp.dot(f, wb_ref[...], precision=_HP)


def _pre(feats, wa, wb):
    R = 2000
    return pl.pallas_call(
        _pre_body,
        grid=(N_NODES // R,),
        in_specs=[
            pl.BlockSpec((R, FE), lambda i: (i, 0)),
            pl.BlockSpec((FE, EH), lambda i: (0, 0)),
            pl.BlockSpec((FE, EH), lambda i: (0, 0)),
        ],
        out_specs=[
            pl.BlockSpec((R, EH), lambda i: (i, 0)),
            pl.BlockSpec((R, EH), lambda i: (i, 0)),
        ],
        out_shape=[jax.ShapeDtypeStruct((N_NODES, EH), jnp.float32)] * 2,
    )(feats, wa, wb)


def _gather(a, b, dst2d, src2d):
    mesh = plsc.VectorSubcoreMesh(core_axis_name="core", subcore_axis_name="subcore")

    @functools.partial(
        pl.kernel,
        out_type=jax.ShapeDtypeStruct((N_EDGES, EH), jnp.float32),
        mesh=mesh,
        compiler_params=pltpu.CompilerParams(use_tc_tiling_on_sc=False),
        scratch_types=[
            pltpu.VMEM((W, EH), jnp.float32),
            pltpu.SemaphoreType.DMA,
            pltpu.SemaphoreType.DMA,
        ],
    )
    def kern(a_hbm, b_hbm, d_hbm, s_hbm, o_hbm, tmp, sem_a, sem_b):
        def body(d_vmem, s_vmem, o_vmem):
            cpa = pltpu.async_copy(a_hbm.at[d_vmem.at[0]], o_vmem, sem_a)
            cpb = pltpu.async_copy(b_hbm.at[s_vmem.at[0]], tmp, sem_b)
            cpa.wait()
            cpb.wait()

            @pl.loop(0, W)
            def _row(r):
                @pl.loop(0, EH, step=16)
                def _col(c):
                    slc = (pl.ds(r, 1), pl.ds(c, 16))
                    o_vmem.at[slc][...] = o_vmem.at[slc][...] + tmp.at[slc][...]

        pltpu.emit_pipeline(
            body,
            grid=(NWIN,),
            in_specs=[
                pl.BlockSpec((1, W), lambda i: (i, 0)),
                pl.BlockSpec((1, W), lambda i: (i, 0)),
            ],
            out_specs=[pl.BlockSpec((W, EH), lambda i: (i, 0))],
            core_axis_name=("core", "subcore"),
            dimension_semantics=(pltpu.PARALLEL,),
        )(d_hbm, s_hbm, o_hbm)

    return kern(a, b, dst2d, src2d)


def _edge_body(eg_ref, ea_ref, wc_ref, b1_ref, w2_ref, b2_ref, o_ref):
    z = eg_ref[...] + jnp.dot(ea_ref[...], wc_ref[...], precision=_HP) + b1_ref[...]
    m1 = _silu(z)
    y = jnp.dot(m1, w2_ref[...], precision=_HP) + b2_ref[...]
    o_ref[...] = _silu(y)


def _edge(eg, ea, wc, b1, w2, b2):
    Be = 4000
    return pl.pallas_call(
        _edge_body,
        grid=(N_EDGES // Be,),
        in_specs=[
            pl.BlockSpec((Be, EH), lambda i: (i, 0)),
            pl.BlockSpec((Be, EA), lambda i: (i, 0)),
            pl.BlockSpec((EA, EH), lambda i: (0, 0)),
            pl.BlockSpec((1, EH), lambda i: (0, 0)),
            pl.BlockSpec((EH, MD), lambda i: (0, 0)),
            pl.BlockSpec((1, MD), lambda i: (0, 0)),
        ],
        out_specs=pl.BlockSpec((Be, MD), lambda i: (i, 0)),
        out_shape=jax.ShapeDtypeStruct((N_EDGES, MD), jnp.float32),
    )(eg, ea, wc, b1, w2, b2)


def _scatter(m2, dst2d):
    mesh = plsc.VectorSubcoreMesh(core_axis_name="core", subcore_axis_name="subcore")

    @functools.partial(
        pl.kernel,
        out_type=jax.ShapeDtypeStruct((2, N_NODES, MD), jnp.float32),
        mesh=mesh,
        compiler_params=pltpu.CompilerParams(use_tc_tiling_on_sc=False),
        scratch_types=[
            pltpu.VMEM((_STRIPE, MD), jnp.float32),
            pltpu.VMEM_SHARED((N_NODES, MD), jnp.float32),
        ],
    )
    def kern(m_hbm, d_hbm, o_hbm, zbuf, acc):
        cid = lax.axis_index("core")
        sid = lax.axis_index("subcore")

        @pl.loop(0, _STRIPE)
        def _z(r):
            zbuf.at[pl.ds(r, 1), pl.ds(0, MD)][...] = jnp.zeros((1, MD), jnp.float32)

        pltpu.sync_copy(zbuf, acc.at[pl.ds(sid * _STRIPE, _STRIPE)])
        plsc.subcore_barrier()

        def body(m_vmem, d_vmem):
            pltpu.sync_copy(m_vmem, acc.at[d_vmem.at[0]], add=True)

        pltpu.emit_pipeline(
            body,
            grid=(NWIN,),
            in_specs=[
                pl.BlockSpec((W, MD), lambda i: (i, 0)),
                pl.BlockSpec((1, W), lambda i: (i, 0)),
            ],
            out_specs=[],
            core_axis_name=("core", "subcore"),
            dimension_semantics=(pltpu.PARALLEL,),
        )(m_hbm, d_hbm)

        plsc.subcore_barrier()
        pltpu.sync_copy(acc.at[pl.ds(sid * _STRIPE, _STRIPE)],
                        o_hbm.at[cid, pl.ds(sid * _STRIPE, _STRIPE)])

    return kern(m2, dst2d)


def _node_body(f_ref, p0_ref, p1_ref, wf_ref, wa_ref, b1_ref, w2_ref, b2_ref, o_ref):
    f = f_ref[...]
    agg = p0_ref[...] + p1_ref[...]
    h = (jnp.dot(f, wf_ref[...], precision=_HP)
         + jnp.dot(agg, wa_ref[...], precision=_HP)
         + b1_ref[...])
    h = _silu(h)
    o_ref[...] = f + jnp.dot(h, w2_ref[...], precision=_HP) + b2_ref[...]


def _node(feats, p0, p1, wf, wa, b1, w2, b2):
    R = 2000
    return pl.pallas_call(
        _node_body,
        grid=(N_NODES // R,),
        in_specs=[
            pl.BlockSpec((R, FE), lambda i: (i, 0)),
            pl.BlockSpec((R, MD), lambda i: (i, 0)),
            pl.BlockSpec((R, MD), lambda i: (i, 0)),
            pl.BlockSpec((FE, NH), lambda i: (0, 0)),
            pl.BlockSpec((MD, NH), lambda i: (0, 0)),
            pl.BlockSpec((1, NH), lambda i: (0, 0)),
            pl.BlockSpec((NH, FE), lambda i: (0, 0)),
            pl.BlockSpec((1, FE), lambda i: (0, 0)),
        ],
        out_specs=pl.BlockSpec((R, FE), lambda i: (i, 0)),
        out_shape=jax.ShapeDtypeStruct((N_NODES, FE), jnp.float32),
    )(feats, p0, p1, wf, wa, b1, w2, b2)


def kernel(x, edge_index, edge_attr, We1, be1, We2, be2, Wn1, bn1, Wn2, bn2):
    feats = x[:, POS:]
    dst2d = edge_index[1].reshape(NWIN, W)
    src2d = edge_index[0].reshape(NWIN, W)

    a, b = _pre(feats, We1[:FE], We1[FE:2 * FE])
    eg = _gather(a, b, dst2d, src2d)
    m2 = _edge(eg, edge_attr, We1[2 * FE:], be1.reshape(1, EH), We2,
               be2.reshape(1, MD))
    parts = _scatter(m2, dst2d)
    feats_out = _node(feats, parts[0], parts[1], Wn1[:FE], Wn1[FE:],
                      bn1.reshape(1, NH), Wn2, bn2.reshape(1, FE))
    return jnp.concatenate([x[:, :POS], feats_out], axis=1)


# speedup vs baseline: 5.1003x; 1.9467x over previous
import functools

import jax
import jax.numpy as jnp
from jax import lax
from jax.experimental import pallas as pl
from jax.experimental.pallas import tpu as pltpu
from jax.experimental.pallas import tpu_sc as plsc

N_NODES = 10000
N_EDGES = 320000
POS = 3
FE = 128
EA = 16
EH = 32
MD = 16
NH = 256

W = 100
NWIN = N_EDGES // W
EP = N_EDGES // 4
_SUB = 16
_STRIPE = N_NODES // _SUB

_HP = lax.Precision.HIGHEST


def _silu(v):
    return v * jax.nn.sigmoid(v)


def _pre_body(f_ref, wa_ref, wb_ref, a_ref, b_ref):
    f = f_ref[...]
    a_ref[...] = jnp.dot(f, wa_ref[...], precision=_HP)
    b_ref[...] = jnp.dot(f, wb_ref[...], precision=_HP)


def _pre(feats, wa, wb):
    R = 2000
    return pl.pallas_call(
        _pre_body,
        grid=(N_NODES // R,),
        in_specs=[
            pl.BlockSpec((R, FE), lambda i: (i, 0)),
            pl.BlockSpec((FE, EH), lambda i: (0, 0)),
            pl.BlockSpec((FE, EH), lambda i: (0, 0)),
        ],
        out_specs=[
            pl.BlockSpec((R, EH), lambda i: (i, 0)),
            pl.BlockSpec((R, EH), lambda i: (i, 0)),
        ],
        out_shape=[jax.ShapeDtypeStruct((N_NODES, EH), jnp.float32)] * 2,
    )(feats, wa, wb)


def _gather(a, b, dst2d, src2d):
    mesh = plsc.VectorSubcoreMesh(core_axis_name="core", subcore_axis_name="subcore")

    @functools.partial(
        pl.kernel,
        out_type=jax.ShapeDtypeStruct((EP, 4 * EH), jnp.float32),
        mesh=mesh,
        compiler_params=pltpu.CompilerParams(use_tc_tiling_on_sc=False),
        scratch_types=[
            pltpu.VMEM((W, EH), jnp.float32),
            pltpu.VMEM((W, EH), jnp.float32),
            pltpu.SemaphoreType.DMA,
            pltpu.SemaphoreType.DMA,
        ],
    )
    def kern(a_hbm, b_hbm, d_hbm, s_hbm, o_hbm, abuf, bbuf, sem_a, sem_b):
        def body(d_vmem, s_vmem, o_vmem):
            cpa = pltpu.async_copy(a_hbm.at[d_vmem.at[0]], abuf, sem_a)
            cpb = pltpu.async_copy(b_hbm.at[s_vmem.at[0]], bbuf, sem_b)
            cpa.wait()
            cpb.wait()

            @pl.loop(0, W // 4)
            def _row(q):
                for l in range(0, 4 * EH, 16):
                    src = (pl.ds(q * 4 + l // EH, 1), pl.ds(l % EH, 16))
                    o_vmem.at[pl.ds(q, 1), pl.ds(l, 16)][...] = (
                        abuf.at[src][...] + bbuf.at[src][...])

        pltpu.emit_pipeline(
            body,
            grid=(NWIN,),
            in_specs=[
                pl.BlockSpec((1, W), lambda i: (i, 0)),
                pl.BlockSpec((1, W), lambda i: (i, 0)),
            ],
            out_specs=[pl.BlockSpec((W // 4, 4 * EH), lambda i: (i, 0))],
            core_axis_name=("core", "subcore"),
            dimension_semantics=(pltpu.PARALLEL,),
        )(d_hbm, s_hbm, o_hbm)

    return kern(a, b, dst2d, src2d)


def _edge_body(eg_ref, ea_ref, wc_ref, b1_ref, w2_ref, b2_ref, o_ref):
    z = eg_ref[...] + jnp.dot(ea_ref[...], wc_ref[...], precision=_HP) + b1_ref[...]
    m1 = _silu(z)
    y = jnp.dot(m1, w2_ref[...], precision=_HP) + b2_ref[...]
    o_ref[...] = _silu(y)


def _edge(eg, ea, wc, b1, w2, b2):
    Be = 8000
    return pl.pallas_call(
        _edge_body,
        grid=(EP // Be,),
        in_specs=[
            pl.BlockSpec((Be, 4 * EH), lambda i: (i, 0)),
            pl.BlockSpec((Be, 4 * EA), lambda i: (i, 0)),
            pl.BlockSpec((4 * EA, 4 * EH), lambda i: (0, 0)),
            pl.BlockSpec((1, 4 * EH), lambda i: (0, 0)),
            pl.BlockSpec((4 * EH, 4 * MD), lambda i: (0, 0)),
            pl.BlockSpec((1, 4 * MD), lambda i: (0, 0)),
        ],
        out_specs=pl.BlockSpec((Be, 4 * MD), lambda i: (i, 0)),
        out_shape=jax.ShapeDtypeStruct((EP, 4 * MD), jnp.float32),
    )(eg, ea, wc, b1, w2, b2)


def _scatter(m2, dst2d):
    mesh = plsc.VectorSubcoreMesh(core_axis_name="core", subcore_axis_name="subcore")

    @functools.partial(
        pl.kernel,
        out_type=jax.ShapeDtypeStruct((2, N_NODES, MD), jnp.float32),
        mesh=mesh,
        compiler_params=pltpu.CompilerParams(use_tc_tiling_on_sc=False),
        scratch_types=[
            pltpu.VMEM((_STRIPE, MD), jnp.float32),
            pltpu.VMEM_SHARED((N_NODES, MD), jnp.float32),
        ],
    )
    def kern(m_hbm, d_hbm, o_hbm, zbuf, acc):
        cid = lax.axis_index("core")
        sid = lax.axis_index("subcore")

        @pl.loop(0, _STRIPE)
        def _z(r):
            zbuf.at[pl.ds(r, 1), pl.ds(0, MD)][...] = jnp.zeros((1, MD), jnp.float32)

        pltpu.sync_copy(zbuf, acc.at[pl.ds(sid * _STRIPE, _STRIPE)])
        plsc.subcore_barrier()

        def body(m_vmem, d_vmem):
            pltpu.sync_copy(m_vmem, acc.at[d_vmem.at[0]], add=True)

        pltpu.emit_pipeline(
            body,
            grid=(NWIN,),
            in_specs=[
                pl.BlockSpec((W, MD), lambda i: (i, 0)),
                pl.BlockSpec((1, W), lambda i: (i, 0)),
            ],
            out_specs=[],
            core_axis_name=("core", "subcore"),
            dimension_semantics=(pltpu.PARALLEL,),
        )(m_hbm, d_hbm)

        plsc.subcore_barrier()
        pltpu.sync_copy(acc.at[pl.ds(sid * _STRIPE, _STRIPE)],
                        o_hbm.at[cid, pl.ds(sid * _STRIPE, _STRIPE)])

    return kern(m2, dst2d)


def _node_body(f_ref, p0_ref, p1_ref, wf_ref, wa_ref, b1_ref, w2_ref, b2_ref, o_ref):
    f = f_ref[...]
    agg = p0_ref[...] + p1_ref[...]
    h = (jnp.dot(f, wf_ref[...], precision=_HP)
         + jnp.dot(agg, wa_ref[...], precision=_HP)
         + b1_ref[...])
    h = _silu(h)
    o_ref[...] = f + jnp.dot(h, w2_ref[...], precision=_HP) + b2_ref[...]


def _node(feats, p0, p1, wf, wa, b1, w2, b2):
    R = 2000
    return pl.pallas_call(
        _node_body,
        grid=(N_NODES // R,),
        in_specs=[
            pl.BlockSpec((R, FE), lambda i: (i, 0)),
            pl.BlockSpec((R, MD), lambda i: (i, 0)),
            pl.BlockSpec((R, MD), lambda i: (i, 0)),
            pl.BlockSpec((FE, NH), lambda i: (0, 0)),
            pl.BlockSpec((MD, NH), lambda i: (0, 0)),
            pl.BlockSpec((1, NH), lambda i: (0, 0)),
            pl.BlockSpec((NH, FE), lambda i: (0, 0)),
            pl.BlockSpec((1, FE), lambda i: (0, 0)),
        ],
        out_specs=pl.BlockSpec((R, FE), lambda i: (i, 0)),
        out_shape=jax.ShapeDtypeStruct((N_NODES, FE), jnp.float32),
    )(feats, p0, p1, wf, wa, b1, w2, b2)


def kernel(x, edge_index, edge_attr, We1, be1, We2, be2, Wn1, bn1, Wn2, bn2):
    feats = x[:, POS:]
    dst2d = edge_index[1].reshape(NWIN, W)
    src2d = edge_index[0].reshape(NWIN, W)

    a, b = _pre(feats, We1[:FE], We1[FE:2 * FE])
    eg = _gather(a, b, dst2d, src2d)
    eye4 = jnp.eye(4, dtype=jnp.float32)
    m2p = _edge(eg, edge_attr.reshape(EP, 4 * EA),
                jnp.kron(eye4, We1[2 * FE:]),
                jnp.tile(be1, 4).reshape(1, 4 * EH),
                jnp.kron(eye4, We2),
                jnp.tile(be2, 4).reshape(1, 4 * MD))
    parts = _scatter(m2p.reshape(N_EDGES, MD), dst2d)
    feats_out = _node(feats, parts[0], parts[1], Wn1[:FE], Wn1[FE:],
                      bn1.reshape(1, NH), Wn2, bn2.reshape(1, FE))
    return jnp.concatenate([x[:, :POS], feats_out], axis=1)


# speedup vs baseline: 5.7960x; 1.1364x over previous
import functools

import jax
import jax.numpy as jnp
from jax import lax
from jax.experimental import pallas as pl
from jax.experimental.pallas import tpu as pltpu
from jax.experimental.pallas import tpu_sc as plsc

N_NODES = 10000
N_EDGES = 320000
POS = 3
FE = 128
EA = 16
EH = 32
MD = 16
NH = 256

W = 100
NWIN = N_EDGES // W
EP = N_EDGES // 4
_SUB = 16
_STRIPE = N_NODES // _SUB

_HP = lax.Precision.HIGHEST


def _silu(v):
    return v * jax.nn.sigmoid(v)


def _pre_body(f_ref, wa_ref, wb_ref, a_ref, b_ref):
    f = f_ref[...]
    a_ref[...] = jnp.dot(f, wa_ref[...], precision=_HP)
    b_ref[...] = jnp.dot(f, wb_ref[...], precision=_HP)


def _pre(feats, wa, wb):
    R = 2000
    return pl.pallas_call(
        _pre_body,
        grid=(N_NODES // R,),
        in_specs=[
            pl.BlockSpec((R, FE), lambda i: (i, 0)),
            pl.BlockSpec((FE, EH), lambda i: (0, 0)),
            pl.BlockSpec((FE, EH), lambda i: (0, 0)),
        ],
        out_specs=[
            pl.BlockSpec((R, EH), lambda i: (i, 0)),
            pl.BlockSpec((R, EH), lambda i: (i, 0)),
        ],
        out_shape=[jax.ShapeDtypeStruct((N_NODES, EH), jnp.float32)] * 2,
    )(feats, wa, wb)


_G = 2
_WPT = NWIN // 32
_NG = _WPT // _G
_OR = _G * W // 4


def _gather(a, b, dst2d, src2d):
    mesh = plsc.VectorSubcoreMesh(core_axis_name="core", subcore_axis_name="subcore")

    @functools.partial(
        pl.kernel,
        out_type=jax.ShapeDtypeStruct((EP, 4 * EH), jnp.float32),
        mesh=mesh,
        compiler_params=pltpu.CompilerParams(use_tc_tiling_on_sc=False),
        scratch_types=[
            pltpu.VMEM((_WPT, W), jnp.int32),
            pltpu.VMEM((_WPT, W), jnp.int32),
            pltpu.VMEM((2, _G * W, EH), jnp.float32),
            pltpu.VMEM((2, _G * W, EH), jnp.float32),
            pltpu.VMEM((2, _OR, 4 * EH), jnp.float32),
            pltpu.SemaphoreType.DMA,
            pltpu.SemaphoreType.DMA,
            pltpu.SemaphoreType.DMA,
            pltpu.SemaphoreType.DMA,
        ],
    )
    def kern(a_hbm, b_hbm, d_hbm, s_hbm, o_hbm, dbuf, sbuf, abuf, bbuf, obuf,
             sg0, sg1, so0, so1):
        cid = lax.axis_index("core")
        sid = lax.axis_index("subcore")
        tid = cid * _SUB + sid
        wbase = tid * _WPT
        obase = wbase * (W // 4)
        sg = (sg0, sg1)
        so = (so0, so1)

        pltpu.sync_copy(d_hbm.at[pl.ds(wbase, _WPT)], dbuf)
        pltpu.sync_copy(s_hbm.at[pl.ds(wbase, _WPT)], sbuf)

        def fire(gg, p):
            for wi in range(_G):
                jj = gg * _G + wi
                dst_sl = abuf.at[p, pl.ds(wi * W, W)]
                pltpu.async_copy(a_hbm.at[dbuf.at[jj]], dst_sl, sg[p])
                dst_sl = bbuf.at[p, pl.ds(wi * W, W)]
                pltpu.async_copy(b_hbm.at[sbuf.at[jj]], dst_sl, sg[p])

        fire(0, 0)
        fire(1, 1)

        @pl.loop(0, _NG // 2)
        def _grp(g):
            for p in range(2):
                gg = g * 2 + p

                @pl.when(g >= 1)
                def _():
                    pltpu.make_async_copy(
                        obuf.at[p], o_hbm.at[pl.ds(0, _OR)], so[p]).wait()

                pltpu.make_async_copy(
                    a_hbm.at[pl.ds(0, _G * W)], abuf.at[p], sg[p]).wait()
                pltpu.make_async_copy(
                    b_hbm.at[pl.ds(0, _G * W)], bbuf.at[p], sg[p]).wait()

                @pl.loop(0, _OR)
                def _row(q):
                    for l in range(0, 4 * EH, 16):
                        src = (p, pl.ds(q * 4 + l // EH, 1), pl.ds(l % EH, 16))
                        obuf.at[p, pl.ds(q, 1), pl.ds(l, 16)][...] = (
                            abuf.at[src][...] + bbuf.at[src][...])

                @pl.when(gg + 2 < _NG)
                def _():
                    fire(gg + 2, p)

                pltpu.async_copy(
                    obuf.at[p], o_hbm.at[pl.ds(obase + gg * _OR, _OR)], so[p])

        for p in range(2):
            pltpu.make_async_copy(
                obuf.at[p], o_hbm.at[pl.ds(0, _OR)], so[p]).wait()

    return kern(a, b, dst2d, src2d)


def _edge_body(eg_ref, ea_ref, wc_ref, b1_ref, w2_ref, b2_ref, o_ref):
    z = eg_ref[...] + jnp.dot(ea_ref[...], wc_ref[...], precision=_HP) + b1_ref[...]
    m1 = _silu(z)
    y = jnp.dot(m1, w2_ref[...], precision=_HP) + b2_ref[...]
    o_ref[...] = _silu(y)


def _edge(eg, ea, wc, b1, w2, b2):
    Be = 8000
    return pl.pallas_call(
        _edge_body,
        grid=(EP // Be,),
        in_specs=[
            pl.BlockSpec((Be, 4 * EH), lambda i: (i, 0)),
            pl.BlockSpec((Be, 4 * EA), lambda i: (i, 0)),
            pl.BlockSpec((4 * EA, 4 * EH), lambda i: (0, 0)),
            pl.BlockSpec((1, 4 * EH), lambda i: (0, 0)),
            pl.BlockSpec((4 * EH, 4 * MD), lambda i: (0, 0)),
            pl.BlockSpec((1, 4 * MD), lambda i: (0, 0)),
        ],
        out_specs=pl.BlockSpec((Be, 4 * MD), lambda i: (i, 0)),
        out_shape=jax.ShapeDtypeStruct((EP, 4 * MD), jnp.float32),
    )(eg, ea, wc, b1, w2, b2)


def _scatter(m2, dst2d):
    mesh = plsc.VectorSubcoreMesh(core_axis_name="core", subcore_axis_name="subcore")

    @functools.partial(
        pl.kernel,
        out_type=jax.ShapeDtypeStruct((2, N_NODES, MD), jnp.float32),
        mesh=mesh,
        compiler_params=pltpu.CompilerParams(use_tc_tiling_on_sc=False),
        scratch_types=[
            pltpu.VMEM((_STRIPE, MD), jnp.float32),
            pltpu.VMEM_SHARED((N_NODES, MD), jnp.float32),
        ],
    )
    def kern(m_hbm, d_hbm, o_hbm, zbuf, acc):
        cid = lax.axis_index("core")
        sid = lax.axis_index("subcore")

        @pl.loop(0, _STRIPE)
        def _z(r):
            zbuf.at[pl.ds(r, 1), pl.ds(0, MD)][...] = jnp.zeros((1, MD), jnp.float32)

        pltpu.sync_copy(zbuf, acc.at[pl.ds(sid * _STRIPE, _STRIPE)])
        plsc.subcore_barrier()

        def body(m_vmem, d_vmem):
            pltpu.sync_copy(m_vmem, acc.at[d_vmem.at[0]], add=True)

        pltpu.emit_pipeline(
            body,
            grid=(NWIN,),
            in_specs=[
                pl.BlockSpec((W, MD), lambda i: (i, 0)),
                pl.BlockSpec((1, W), lambda i: (i, 0)),
            ],
            out_specs=[],
            core_axis_name=("core", "subcore"),
            dimension_semantics=(pltpu.PARALLEL,),
        )(m_hbm, d_hbm)

        plsc.subcore_barrier()
        pltpu.sync_copy(acc.at[pl.ds(sid * _STRIPE, _STRIPE)],
                        o_hbm.at[cid, pl.ds(sid * _STRIPE, _STRIPE)])

    return kern(m2, dst2d)


def _node_body(f_ref, p0_ref, p1_ref, wf_ref, wa_ref, b1_ref, w2_ref, b2_ref, o_ref):
    f = f_ref[...]
    agg = p0_ref[...] + p1_ref[...]
    h = (jnp.dot(f, wf_ref[...], precision=_HP)
         + jnp.dot(agg, wa_ref[...], precision=_HP)
         + b1_ref[...])
    h = _silu(h)
    o_ref[...] = f + jnp.dot(h, w2_ref[...], precision=_HP) + b2_ref[...]


def _node(feats, p0, p1, wf, wa, b1, w2, b2):
    R = 2000
    return pl.pallas_call(
        _node_body,
        grid=(N_NODES // R,),
        in_specs=[
            pl.BlockSpec((R, FE), lambda i: (i, 0)),
            pl.BlockSpec((R, MD), lambda i: (i, 0)),
            pl.BlockSpec((R, MD), lambda i: (i, 0)),
            pl.BlockSpec((FE, NH), lambda i: (0, 0)),
            pl.BlockSpec((MD, NH), lambda i: (0, 0)),
            pl.BlockSpec((1, NH), lambda i: (0, 0)),
            pl.BlockSpec((NH, FE), lambda i: (0, 0)),
            pl.BlockSpec((1, FE), lambda i: (0, 0)),
        ],
        out_specs=pl.BlockSpec((R, FE), lambda i: (i, 0)),
        out_shape=jax.ShapeDtypeStruct((N_NODES, FE), jnp.float32),
    )(feats, p0, p1, wf, wa, b1, w2, b2)


def kernel(x, edge_index, edge_attr, We1, be1, We2, be2, Wn1, bn1, Wn2, bn2):
    feats = x[:, POS:]
    dst2d = edge_index[1].reshape(NWIN, W)
    src2d = edge_index[0].reshape(NWIN, W)

    a, b = _pre(feats, We1[:FE], We1[FE:2 * FE])
    eg = _gather(a, b, dst2d, src2d)
    eye4 = jnp.eye(4, dtype=jnp.float32)
    m2p = _edge(eg, edge_attr.reshape(EP, 4 * EA),
                jnp.kron(eye4, We1[2 * FE:]),
                jnp.tile(be1, 4).reshape(1, 4 * EH),
                jnp.kron(eye4, We2),
                jnp.tile(be2, 4).reshape(1, 4 * MD))
    parts = _scatter(m2p.reshape(N_EDGES, MD), dst2d)
    feats_out = _node(feats, parts[0], parts[1], Wn1[:FE], Wn1[FE:],
                      bn1.reshape(1, NH), Wn2, bn2.reshape(1, FE))
    return jnp.concatenate([x[:, :POS], feats_out], axis=1)


# speedup vs baseline: 6.8978x; 1.1901x over previous
import functools

import jax
import jax.numpy as jnp
from jax import lax
from jax.experimental import pallas as pl
from jax.experimental.pallas import tpu as pltpu
from jax.experimental.pallas import tpu_sc as plsc

N_NODES = 10000
N_EDGES = 320000
POS = 3
FE = 128
EA = 16
EH = 32
MD = 16
NH = 256

W = 100
NWIN = N_EDGES // W
EP = N_EDGES // 4
W_S = 80
NWIN_S = N_EDGES // W_S
_SUB = 16
_STRIPE = N_NODES // _SUB

_HP = lax.Precision.HIGHEST


def _silu(v):
    return v * jax.nn.sigmoid(v)


def _pre_body(f_ref, wa_ref, wb_ref, a_ref, b_ref):
    f = f_ref[...]
    a_ref[...] = jnp.dot(f, wa_ref[...], precision=_HP)
    b_ref[...] = jnp.dot(f, wb_ref[...], precision=_HP)


def _pre(feats, wa, wb):
    R = 2000
    return pl.pallas_call(
        _pre_body,
        grid=(N_NODES // R,),
        in_specs=[
            pl.BlockSpec((R, FE), lambda i: (i, 0)),
            pl.BlockSpec((FE, EH), lambda i: (0, 0)),
            pl.BlockSpec((FE, EH), lambda i: (0, 0)),
        ],
        out_specs=[
            pl.BlockSpec((R, EH), lambda i: (i, 0)),
            pl.BlockSpec((R, EH), lambda i: (i, 0)),
        ],
        out_shape=[jax.ShapeDtypeStruct((N_NODES, EH), jnp.float32)] * 2,
    )(feats, wa, wb)


_G = 2
_WPT = NWIN // 32
_NG = _WPT // _G
_OR = _G * W // 4


def _gather(a, b, dst2d, src2d):
    mesh = plsc.VectorSubcoreMesh(core_axis_name="core", subcore_axis_name="subcore")

    @functools.partial(
        pl.kernel,
        out_type=jax.ShapeDtypeStruct((EP, 4 * EH), jnp.float32),
        mesh=mesh,
        compiler_params=pltpu.CompilerParams(use_tc_tiling_on_sc=False),
        scratch_types=[
            pltpu.VMEM((_WPT, W), jnp.int32),
            pltpu.VMEM((_WPT, W), jnp.int32),
            pltpu.VMEM((2, _G * W, EH), jnp.float32),
            pltpu.VMEM((2, _G * W, EH), jnp.float32),
            pltpu.VMEM((2, _OR, 4 * EH), jnp.float32),
            pltpu.SemaphoreType.DMA,
            pltpu.SemaphoreType.DMA,
            pltpu.SemaphoreType.DMA,
            pltpu.SemaphoreType.DMA,
        ],
    )
    def kern(a_hbm, b_hbm, d_hbm, s_hbm, o_hbm, dbuf, sbuf, abuf, bbuf, obuf,
             sg0, sg1, so0, so1):
        cid = lax.axis_index("core")
        sid = lax.axis_index("subcore")
        tid = cid * _SUB + sid
        wbase = tid * _WPT
        obase = wbase * (W // 4)
        sg = (sg0, sg1)
        so = (so0, so1)

        pltpu.sync_copy(d_hbm.at[pl.ds(wbase, _WPT)], dbuf)
        pltpu.sync_copy(s_hbm.at[pl.ds(wbase, _WPT)], sbuf)

        def fire(gg, p):
            for wi in range(_G):
                jj = gg * _G + wi
                dst_sl = abuf.at[p, pl.ds(wi * W, W)]
                pltpu.async_copy(a_hbm.at[dbuf.at[jj]], dst_sl, sg[p])
                dst_sl = bbuf.at[p, pl.ds(wi * W, W)]
                pltpu.async_copy(b_hbm.at[sbuf.at[jj]], dst_sl, sg[p])

        fire(0, 0)
        fire(1, 1)

        @pl.loop(0, _NG // 2)
        def _grp(g):
            for p in range(2):
                gg = g * 2 + p

                @pl.when(g >= 1)
                def _():
                    pltpu.make_async_copy(
                        obuf.at[p], o_hbm.at[pl.ds(0, _OR)], so[p]).wait()

                pltpu.make_async_copy(
                    a_hbm.at[pl.ds(0, _G * W)], abuf.at[p], sg[p]).wait()
                pltpu.make_async_copy(
                    b_hbm.at[pl.ds(0, _G * W)], bbuf.at[p], sg[p]).wait()

                @pl.loop(0, _OR)
                def _row(q):
                    for l in range(0, 4 * EH, 16):
                        src = (p, pl.ds(q * 4 + l // EH, 1), pl.ds(l % EH, 16))
                        obuf.at[p, pl.ds(q, 1), pl.ds(l, 16)][...] = (
                            abuf.at[src][...] + bbuf.at[src][...])

                @pl.when(gg + 2 < _NG)
                def _():
                    fire(gg + 2, p)

                pltpu.async_copy(
                    obuf.at[p], o_hbm.at[pl.ds(obase + gg * _OR, _OR)], so[p])

        for p in range(2):
            pltpu.make_async_copy(
                obuf.at[p], o_hbm.at[pl.ds(0, _OR)], so[p]).wait()

    return kern(a, b, dst2d, src2d)


def _edge_body(eg_ref, ea_ref, wc_ref, b1_ref, w2_ref, b2_ref, o_ref):
    z = eg_ref[...] + jnp.dot(ea_ref[...], wc_ref[...]) + b1_ref[...]
    m1 = _silu(z)
    y = jnp.dot(m1, w2_ref[...]) + b2_ref[...]
    o_ref[...] = _silu(y)


def _edge(eg, ea, wc, b1, w2, b2):
    Be = 8000
    return pl.pallas_call(
        _edge_body,
        grid=(EP // Be,),
        in_specs=[
            pl.BlockSpec((Be, 4 * EH), lambda i: (i, 0)),
            pl.BlockSpec((Be, 4 * EA), lambda i: (i, 0)),
            pl.BlockSpec((4 * EA, 4 * EH), lambda i: (0, 0)),
            pl.BlockSpec((1, 4 * EH), lambda i: (0, 0)),
            pl.BlockSpec((4 * EH, 4 * MD), lambda i: (0, 0)),
            pl.BlockSpec((1, 4 * MD), lambda i: (0, 0)),
        ],
        out_specs=pl.BlockSpec((Be, 4 * MD), lambda i: (i, 0)),
        out_shape=jax.ShapeDtypeStruct((EP, 4 * MD), jnp.float32),
    )(eg, ea, wc, b1, w2, b2)


def _scatter(m2, dst2d):
    mesh = plsc.VectorSubcoreMesh(core_axis_name="core", subcore_axis_name="subcore")

    @functools.partial(
        pl.kernel,
        out_type=jax.ShapeDtypeStruct((2, N_NODES, MD), jnp.float32),
        mesh=mesh,
        compiler_params=pltpu.CompilerParams(use_tc_tiling_on_sc=False),
        scratch_types=[
            pltpu.VMEM((_STRIPE, MD), jnp.float32),
            pltpu.VMEM_SHARED((N_NODES, MD), jnp.float32),
        ],
    )
    def kern(m_hbm, d_hbm, o_hbm, zbuf, acc):
        cid = lax.axis_index("core")
        sid = lax.axis_index("subcore")

        @pl.loop(0, _STRIPE)
        def _z(r):
            zbuf.at[pl.ds(r, 1), pl.ds(0, MD)][...] = jnp.zeros((1, MD), jnp.float32)

        pltpu.sync_copy(zbuf, acc.at[pl.ds(sid * _STRIPE, _STRIPE)])
        plsc.subcore_barrier()

        def body(m_vmem, d_vmem):
            pltpu.sync_copy(m_vmem, acc.at[d_vmem.at[0]], add=True)

        pltpu.emit_pipeline(
            body,
            grid=(NWIN_S,),
            in_specs=[
                pl.BlockSpec((W_S, MD), lambda i: (i, 0)),
                pl.BlockSpec((1, W_S), lambda i: (i, 0)),
            ],
            out_specs=[],
            core_axis_name=("core", "subcore"),
            dimension_semantics=(pltpu.PARALLEL,),
        )(m_hbm, d_hbm)

        plsc.subcore_barrier()
        pltpu.sync_copy(acc.at[pl.ds(sid * _STRIPE, _STRIPE)],
                        o_hbm.at[cid, pl.ds(sid * _STRIPE, _STRIPE)])

    return kern(m2, dst2d)


def _node_body(f_ref, p0_ref, p1_ref, wf_ref, wa_ref, b1_ref, w2_ref, b2_ref, o_ref):
    f = f_ref[...]
    agg = p0_ref[...] + p1_ref[...]
    h = (jnp.dot(f, wf_ref[...], precision=_HP)
         + jnp.dot(agg, wa_ref[...], precision=_HP)
         + b1_ref[...])
    h = _silu(h)
    o_ref[...] = f + jnp.dot(h, w2_ref[...], precision=_HP) + b2_ref[...]


def _node(feats, p0, p1, wf, wa, b1, w2, b2):
    R = 2000
    return pl.pallas_call(
        _node_body,
        grid=(N_NODES // R,),
        in_specs=[
            pl.BlockSpec((R, FE), lambda i: (i, 0)),
            pl.BlockSpec((R, MD), lambda i: (i, 0)),
            pl.BlockSpec((R, MD), lambda i: (i, 0)),
            pl.BlockSpec((FE, NH), lambda i: (0, 0)),
            pl.BlockSpec((MD, NH), lambda i: (0, 0)),
            pl.BlockSpec((1, NH), lambda i: (0, 0)),
            pl.BlockSpec((NH, FE), lambda i: (0, 0)),
            pl.BlockSpec((1, FE), lambda i: (0, 0)),
        ],
        out_specs=pl.BlockSpec((R, FE), lambda i: (i, 0)),
        out_shape=jax.ShapeDtypeStruct((N_NODES, FE), jnp.float32),
    )(feats, p0, p1, wf, wa, b1, w2, b2)


def kernel(x, edge_index, edge_attr, We1, be1, We2, be2, Wn1, bn1, Wn2, bn2):
    feats = x[:, POS:]
    dst2d = edge_index[1].reshape(NWIN, W)
    src2d = edge_index[0].reshape(NWIN, W)

    a, b = _pre(feats, We1[:FE], We1[FE:2 * FE])
    eg = _gather(a, b, dst2d, src2d)
    eye4 = jnp.eye(4, dtype=jnp.float32)
    m2p = _edge(eg, edge_attr.reshape(EP, 4 * EA),
                jnp.kron(eye4, We1[2 * FE:]),
                jnp.tile(be1, 4).reshape(1, 4 * EH),
                jnp.kron(eye4, We2),
                jnp.tile(be2, 4).reshape(1, 4 * MD))
    parts = _scatter(m2p.reshape(N_EDGES, MD),
                     edge_index[1].reshape(NWIN_S, W_S))
    feats_out = _node(feats, parts[0], parts[1], Wn1[:FE], Wn1[FE:],
                      bn1.reshape(1, NH), Wn2, bn2.reshape(1, FE))
    return jnp.concatenate([x[:, :POS], feats_out], axis=1)


# speedup vs baseline: 7.4802x; 1.0844x over previous
import functools

import jax
import jax.numpy as jnp
from jax import lax
from jax.experimental import pallas as pl
from jax.experimental.pallas import tpu as pltpu
from jax.experimental.pallas import tpu_sc as plsc

N_NODES = 10000
N_EDGES = 320000
POS = 3
FE = 128
EA = 16
EH = 32
MD = 16
NH = 256

W = 100
NWIN = N_EDGES // W
EP = N_EDGES // 4
W_S = 80
NWIN_S = N_EDGES // W_S
_SUB = 16
_STRIPE = N_NODES // _SUB

def _silu(v):
    return v * jax.nn.sigmoid(v)


def _pre_body(f_ref, wa_ref, wb_ref, a_ref, b_ref):
    f = f_ref[...]
    a_ref[...] = jnp.dot(f, wa_ref[...])
    b_ref[...] = jnp.dot(f, wb_ref[...])


def _pre(feats, wa, wb):
    R = 2000
    return pl.pallas_call(
        _pre_body,
        grid=(N_NODES // R,),
        in_specs=[
            pl.BlockSpec((R, FE), lambda i: (i, 0)),
            pl.BlockSpec((FE, EH), lambda i: (0, 0)),
            pl.BlockSpec((FE, EH), lambda i: (0, 0)),
        ],
        out_specs=[
            pl.BlockSpec((R, EH), lambda i: (i, 0)),
            pl.BlockSpec((R, EH), lambda i: (i, 0)),
        ],
        out_shape=[jax.ShapeDtypeStruct((N_NODES, EH), jnp.float32)] * 2,
    )(feats, wa, wb)


_G = 2
_WPT = NWIN // 32
_NG = _WPT // _G
_OR = _G * W // 4


def _gather(a, b, dst2d, src2d):
    mesh = plsc.VectorSubcoreMesh(core_axis_name="core", subcore_axis_name="subcore")

    @functools.partial(
        pl.kernel,
        out_type=jax.ShapeDtypeStruct((EP, 4 * EH), jnp.float32),
        mesh=mesh,
        compiler_params=pltpu.CompilerParams(use_tc_tiling_on_sc=False),
        scratch_types=[
            pltpu.VMEM((_WPT, W), jnp.int32),
            pltpu.VMEM((_WPT, W), jnp.int32),
            pltpu.VMEM((2, _G * W, EH), jnp.float32),
            pltpu.VMEM((2, _G * W, EH), jnp.float32),
            pltpu.VMEM((2, _OR, 4 * EH), jnp.float32),
            pltpu.SemaphoreType.DMA,
            pltpu.SemaphoreType.DMA,
            pltpu.SemaphoreType.DMA,
            pltpu.SemaphoreType.DMA,
        ],
    )
    def kern(a_hbm, b_hbm, d_hbm, s_hbm, o_hbm, dbuf, sbuf, abuf, bbuf, obuf,
             sg0, sg1, so0, so1):
        cid = lax.axis_index("core")
        sid = lax.axis_index("subcore")
        tid = cid * _SUB + sid
        wbase = tid * _WPT
        obase = wbase * (W // 4)
        sg = (sg0, sg1)
        so = (so0, so1)

        pltpu.sync_copy(d_hbm.at[pl.ds(wbase, _WPT)], dbuf)
        pltpu.sync_copy(s_hbm.at[pl.ds(wbase, _WPT)], sbuf)

        def fire(gg, p):
            for wi in range(_G):
                jj = gg * _G + wi
                dst_sl = abuf.at[p, pl.ds(wi * W, W)]
                pltpu.async_copy(a_hbm.at[dbuf.at[jj]], dst_sl, sg[p])
                dst_sl = bbuf.at[p, pl.ds(wi * W, W)]
                pltpu.async_copy(b_hbm.at[sbuf.at[jj]], dst_sl, sg[p])

        fire(0, 0)
        fire(1, 1)

        @pl.loop(0, _NG // 2)
        def _grp(g):
            for p in range(2):
                gg = g * 2 + p

                @pl.when(g >= 1)
                def _():
                    pltpu.make_async_copy(
                        obuf.at[p], o_hbm.at[pl.ds(0, _OR)], so[p]).wait()

                pltpu.make_async_copy(
                    a_hbm.at[pl.ds(0, _G * W)], abuf.at[p], sg[p]).wait()
                pltpu.make_async_copy(
                    b_hbm.at[pl.ds(0, _G * W)], bbuf.at[p], sg[p]).wait()

                @pl.loop(0, _OR)
                def _row(q):
                    for l in range(0, 4 * EH, 16):
                        src = (p, pl.ds(q * 4 + l // EH, 1), pl.ds(l % EH, 16))
                        obuf.at[p, pl.ds(q, 1), pl.ds(l, 16)][...] = (
                            abuf.at[src][...] + bbuf.at[src][...])

                @pl.when(gg + 2 < _NG)
                def _():
                    fire(gg + 2, p)

                pltpu.async_copy(
                    obuf.at[p], o_hbm.at[pl.ds(obase + gg * _OR, _OR)], so[p])

        for p in range(2):
            pltpu.make_async_copy(
                obuf.at[p], o_hbm.at[pl.ds(0, _OR)], so[p]).wait()

    return kern(a, b, dst2d, src2d)


def _edge_body(eg_ref, ea_ref, wc_ref, b1_ref, w2_ref, b2_ref, o_ref):
    z = eg_ref[...] + jnp.dot(ea_ref[...], wc_ref[...]) + b1_ref[...]
    m1 = _silu(z)
    y = jnp.dot(m1, w2_ref[...]) + b2_ref[...]
    o_ref[...] = _silu(y)


def _edge(eg, ea, wc, b1, w2, b2):
    Be = 8000
    return pl.pallas_call(
        _edge_body,
        grid=(EP // Be,),
        in_specs=[
            pl.BlockSpec((Be, 4 * EH), lambda i: (i, 0)),
            pl.BlockSpec((Be, 4 * EA), lambda i: (i, 0)),
            pl.BlockSpec((4 * EA, 4 * EH), lambda i: (0, 0)),
            pl.BlockSpec((1, 4 * EH), lambda i: (0, 0)),
            pl.BlockSpec((4 * EH, 4 * MD), lambda i: (0, 0)),
            pl.BlockSpec((1, 4 * MD), lambda i: (0, 0)),
        ],
        out_specs=pl.BlockSpec((Be, 4 * MD), lambda i: (i, 0)),
        out_shape=jax.ShapeDtypeStruct((EP, 4 * MD), jnp.float32),
    )(eg, ea, wc, b1, w2, b2)


def _scatter(m2, dst2d):
    mesh = plsc.VectorSubcoreMesh(core_axis_name="core", subcore_axis_name="subcore")

    @functools.partial(
        pl.kernel,
        out_type=jax.ShapeDtypeStruct((2, N_NODES, MD), jnp.float32),
        mesh=mesh,
        compiler_params=pltpu.CompilerParams(use_tc_tiling_on_sc=False),
        scratch_types=[
            pltpu.VMEM((_STRIPE, MD), jnp.float32),
            pltpu.VMEM_SHARED((N_NODES, MD), jnp.float32),
        ],
    )
    def kern(m_hbm, d_hbm, o_hbm, zbuf, acc):
        cid = lax.axis_index("core")
        sid = lax.axis_index("subcore")

        @pl.loop(0, _STRIPE)
        def _z(r):
            zbuf.at[pl.ds(r, 1), pl.ds(0, MD)][...] = jnp.zeros((1, MD), jnp.float32)

        pltpu.sync_copy(zbuf, acc.at[pl.ds(sid * _STRIPE, _STRIPE)])
        plsc.subcore_barrier()

        def body(m_vmem, d_vmem):
            pltpu.sync_copy(m_vmem, acc.at[d_vmem.at[0]], add=True)

        pltpu.emit_pipeline(
            body,
            grid=(NWIN_S,),
            in_specs=[
                pl.BlockSpec((W_S, MD), lambda i: (i, 0)),
                pl.BlockSpec((1, W_S), lambda i: (i, 0)),
            ],
            out_specs=[],
            core_axis_name=("core", "subcore"),
            dimension_semantics=(pltpu.PARALLEL,),
        )(m_hbm, d_hbm)

        plsc.subcore_barrier()
        pltpu.sync_copy(acc.at[pl.ds(sid * _STRIPE, _STRIPE)],
                        o_hbm.at[cid, pl.ds(sid * _STRIPE, _STRIPE)])

    return kern(m2, dst2d)


def _node_body(f_ref, p0_ref, p1_ref, wf_ref, wa_ref, b1_ref, w2_ref, b2_ref, o_ref):
    f = f_ref[...]
    agg = p0_ref[...] + p1_ref[...]
    h = (jnp.dot(f, wf_ref[...])
         + jnp.dot(agg, wa_ref[...])
         + b1_ref[...])
    h = _silu(h)
    o_ref[...] = f + jnp.dot(h, w2_ref[...]) + b2_ref[...]


def _node(feats, p0, p1, wf, wa, b1, w2, b2):
    R = 2000
    return pl.pallas_call(
        _node_body,
        grid=(N_NODES // R,),
        in_specs=[
            pl.BlockSpec((R, FE), lambda i: (i, 0)),
            pl.BlockSpec((R, MD), lambda i: (i, 0)),
            pl.BlockSpec((R, MD), lambda i: (i, 0)),
            pl.BlockSpec((FE, NH), lambda i: (0, 0)),
            pl.BlockSpec((MD, NH), lambda i: (0, 0)),
            pl.BlockSpec((1, NH), lambda i: (0, 0)),
            pl.BlockSpec((NH, FE), lambda i: (0, 0)),
            pl.BlockSpec((1, FE), lambda i: (0, 0)),
        ],
        out_specs=pl.BlockSpec((R, FE), lambda i: (i, 0)),
        out_shape=jax.ShapeDtypeStruct((N_NODES, FE), jnp.float32),
    )(feats, p0, p1, wf, wa, b1, w2, b2)


def kernel(x, edge_index, edge_attr, We1, be1, We2, be2, Wn1, bn1, Wn2, bn2):
    feats = x[:, POS:]
    dst2d = edge_index[1].reshape(NWIN, W)
    src2d = edge_index[0].reshape(NWIN, W)

    a, b = _pre(feats, We1[:FE], We1[FE:2 * FE])
    eg = _gather(a, b, dst2d, src2d)
    eye4 = jnp.eye(4, dtype=jnp.float32)
    m2p = _edge(eg, edge_attr.reshape(EP, 4 * EA),
                jnp.kron(eye4, We1[2 * FE:]),
                jnp.tile(be1, 4).reshape(1, 4 * EH),
                jnp.kron(eye4, We2),
                jnp.tile(be2, 4).reshape(1, 4 * MD))
    parts = _scatter(m2p.reshape(N_EDGES, MD),
                     edge_index[1].reshape(NWIN_S, W_S))
    feats_out = _node(feats, parts[0], parts[1], Wn1[:FE], Wn1[FE:],
                      bn1.reshape(1, NH), Wn2, bn2.reshape(1, FE))
    return jnp.concatenate([x[:, :POS], feats_out], axis=1)


# speedup vs baseline: 7.8126x; 1.0444x over previous
import functools

import jax
import jax.numpy as jnp
from jax import lax
from jax.experimental import pallas as pl
from jax.experimental.pallas import tpu as pltpu
from jax.experimental.pallas import tpu_sc as plsc

N_NODES = 10000
N_EDGES = 320000
POS = 3
FE = 128
EA = 16
EH = 32
MD = 16
NH = 256

W = 80
EPT = N_EDGES // 32
EP = N_EDGES // 4
_SUB = 16
_STRIPE = N_NODES // _SUB

def _silu(v):
    return v * jax.nn.sigmoid(v)


def _pre_body(f_ref, wa_ref, wb_ref, a_ref, b_ref):
    f = f_ref[...]
    a_ref[...] = jnp.dot(f, wa_ref[...])
    b_ref[...] = jnp.dot(f, wb_ref[...])


def _pre(feats, wa, wb):
    R = 2000
    return pl.pallas_call(
        _pre_body,
        grid=(N_NODES // R,),
        in_specs=[
            pl.BlockSpec((R, FE), lambda i: (i, 0)),
            pl.BlockSpec((FE, EH), lambda i: (0, 0)),
            pl.BlockSpec((FE, EH), lambda i: (0, 0)),
        ],
        out_specs=[
            pl.BlockSpec((R, EH), lambda i: (i, 0)),
            pl.BlockSpec((R, EH), lambda i: (i, 0)),
        ],
        out_shape=[jax.ShapeDtypeStruct((N_NODES, EH), jnp.float32)] * 2,
    )(feats, wa, wb)


_G = 5
_WPT = EPT // W
_NG = _WPT // _G
_GW = _G * W
_OR = _GW // 4


def _gather(a, b, dst1, src1):
    mesh = plsc.VectorSubcoreMesh(core_axis_name="core", subcore_axis_name="subcore")

    @functools.partial(
        pl.kernel,
        out_type=jax.ShapeDtypeStruct((EP, 4 * EH), jnp.float32),
        mesh=mesh,
        compiler_params=pltpu.CompilerParams(use_tc_tiling_on_sc=False),
        scratch_types=[
            pltpu.VMEM((EPT,), jnp.int32),
            pltpu.VMEM((EPT,), jnp.int32),
            pltpu.VMEM((2, _GW, EH), jnp.float32),
            pltpu.VMEM((2, _GW, EH), jnp.float32),
            pltpu.VMEM((2, _OR, 4 * EH), jnp.float32),
            pltpu.SemaphoreType.DMA,
            pltpu.SemaphoreType.DMA,
            pltpu.SemaphoreType.DMA,
            pltpu.SemaphoreType.DMA,
        ],
    )
    def kern(a_hbm, b_hbm, d_hbm, s_hbm, o_hbm, dbuf, sbuf, abuf, bbuf, obuf,
             sg0, sg1, so0, so1):
        cid = lax.axis_index("core")
        sid = lax.axis_index("subcore")
        tid = cid * _SUB + sid
        ebase = tid * EPT
        obase = tid * (EPT // 4)
        sg = (sg0, sg1)
        so = (so0, so1)

        pltpu.sync_copy(d_hbm.at[pl.ds(ebase, EPT)], dbuf)
        pltpu.sync_copy(s_hbm.at[pl.ds(ebase, EPT)], sbuf)

        def fire(gg, p):
            for wi in range(_G):
                off = gg * _GW + wi * W
                idx_d = dbuf.at[pl.ds(off, W)]
                idx_s = sbuf.at[pl.ds(off, W)]
                pltpu.async_copy(a_hbm.at[idx_d],
                                 abuf.at[p, pl.ds(wi * W, W)], sg[p])
                pltpu.async_copy(b_hbm.at[idx_s],
                                 bbuf.at[p, pl.ds(wi * W, W)], sg[p])

        def reclaim(p):
            pltpu.make_async_copy(
                obuf.at[p], o_hbm.at[pl.ds(0, _OR)], so[p]).wait()

        def process(gg, p, prefetch):
            pltpu.make_async_copy(
                a_hbm.at[pl.ds(0, _GW)], abuf.at[p], sg[p]).wait()
            pltpu.make_async_copy(
                b_hbm.at[pl.ds(0, _GW)], bbuf.at[p], sg[p]).wait()

            @pl.loop(0, _OR)
            def _row(q):
                for l in range(0, 4 * EH, 16):
                    src = (p, pl.ds(q * 4 + l // EH, 1), pl.ds(l % EH, 16))
                    obuf.at[p, pl.ds(q, 1), pl.ds(l, 16)][...] = (
                        abuf.at[src][...] + bbuf.at[src][...])

            if prefetch:
                @pl.when(gg + 2 < _NG)
                def _():
                    fire(gg + 2, p)

            pltpu.async_copy(
                obuf.at[p], o_hbm.at[pl.ds(obase + gg * _OR, _OR)], so[p])

        fire(0, 0)
        fire(1, 1)

        @pl.loop(0, (_NG - 1) // 2)
        def _grp(g):
            for p in range(2):
                @pl.when(g >= 1)
                def _():
                    reclaim(p)
                process(g * 2 + p, p, True)

        reclaim(0)
        process(_NG - 1, 0, False)
        reclaim(1)
        reclaim(0)

    return kern(a, b, dst1, src1)


def _edge_body(eg_ref, ea_ref, wc_ref, b1_ref, w2_ref, b2_ref, o_ref):
    z = eg_ref[...] + jnp.dot(ea_ref[...], wc_ref[...]) + b1_ref[...]
    m1 = _silu(z)
    y = jnp.dot(m1, w2_ref[...]) + b2_ref[...]
    o_ref[...] = _silu(y)


def _edge(eg, ea, wc, b1, w2, b2):
    Be = 8000
    return pl.pallas_call(
        _edge_body,
        grid=(EP // Be,),
        in_specs=[
            pl.BlockSpec((Be, 4 * EH), lambda i: (i, 0)),
            pl.BlockSpec((Be, 4 * EA), lambda i: (i, 0)),
            pl.BlockSpec((4 * EA, 4 * EH), lambda i: (0, 0)),
            pl.BlockSpec((1, 4 * EH), lambda i: (0, 0)),
            pl.BlockSpec((4 * EH, 4 * MD), lambda i: (0, 0)),
            pl.BlockSpec((1, 4 * MD), lambda i: (0, 0)),
        ],
        out_specs=pl.BlockSpec((Be, 4 * MD), lambda i: (i, 0)),
        out_shape=jax.ShapeDtypeStruct((EP, 4 * MD), jnp.float32),
    )(eg, ea, wc, b1, w2, b2)


def _scatter(m2, dst1):
    mesh = plsc.VectorSubcoreMesh(core_axis_name="core", subcore_axis_name="subcore")

    @functools.partial(
        pl.kernel,
        out_type=jax.ShapeDtypeStruct((2, N_NODES, MD), jnp.float32),
        mesh=mesh,
        compiler_params=pltpu.CompilerParams(use_tc_tiling_on_sc=False),
        scratch_types=[
            pltpu.VMEM((_STRIPE, MD), jnp.float32),
            pltpu.VMEM_SHARED((N_NODES, MD), jnp.float32),
            pltpu.VMEM((2, W), jnp.int32),
            pltpu.VMEM((2, W, MD), jnp.float32),
            pltpu.SemaphoreType.DMA,
            pltpu.SemaphoreType.DMA,
        ],
    )
    def kern(m_hbm, d_hbm, o_hbm, zbuf, acc, ibuf, mbuf, sm0, sm1):
        cid = lax.axis_index("core")
        sid = lax.axis_index("subcore")
        tid = cid * _SUB + sid
        ebase = tid * EPT
        sm = (sm0, sm1)

        @pl.loop(0, _STRIPE)
        def _z(r):
            zbuf.at[pl.ds(r, 1), pl.ds(0, MD)][...] = jnp.zeros((1, MD), jnp.float32)

        pltpu.sync_copy(zbuf, acc.at[pl.ds(sid * _STRIPE, _STRIPE)])
        plsc.subcore_barrier()

        def fire(j, p):
            pltpu.async_copy(d_hbm.at[pl.ds(ebase + j * W, W)],
                             ibuf.at[p], sm[p])
            pltpu.async_copy(m_hbm.at[pl.ds(ebase + j * W, W)],
                             mbuf.at[p], sm[p])

        def process(j, p, prefetch):
            pltpu.make_async_copy(
                d_hbm.at[pl.ds(0, W)], ibuf.at[p], sm[p]).wait()
            pltpu.make_async_copy(
                m_hbm.at[pl.ds(0, W)], mbuf.at[p], sm[p]).wait()
            if prefetch:
                @pl.when(j + 2 < _WPT)
                def _():
                    fire(j + 2, p)
            pltpu.sync_copy(mbuf.at[p], acc.at[ibuf.at[p]], add=True)

        fire(0, 0)
        fire(1, 1)

        @pl.loop(0, (_WPT - 1) // 2)
        def _win(g):
            for p in range(2):
                process(g * 2 + p, p, True)

        process(_WPT - 1, 0, False)

        plsc.subcore_barrier()
        pltpu.sync_copy(acc.at[pl.ds(sid * _STRIPE, _STRIPE)],
                        o_hbm.at[cid, pl.ds(sid * _STRIPE, _STRIPE)])

    return kern(m2, dst1)


def _node_body(f_ref, p0_ref, p1_ref, wf_ref, wa_ref, b1_ref, w2_ref, b2_ref, o_ref):
    f = f_ref[...]
    agg = p0_ref[...] + p1_ref[...]
    h = (jnp.dot(f, wf_ref[...])
         + jnp.dot(agg, wa_ref[...])
         + b1_ref[...])
    h = _silu(h)
    o_ref[...] = f + jnp.dot(h, w2_ref[...]) + b2_ref[...]


def _node(feats, p0, p1, wf, wa, b1, w2, b2):
    R = 2000
    return pl.pallas_call(
        _node_body,
        grid=(N_NODES // R,),
        in_specs=[
            pl.BlockSpec((R, FE), lambda i: (i, 0)),
            pl.BlockSpec((R, MD), lambda i: (i, 0)),
            pl.BlockSpec((R, MD), lambda i: (i, 0)),
            pl.BlockSpec((FE, NH), lambda i: (0, 0)),
            pl.BlockSpec((MD, NH), lambda i: (0, 0)),
            pl.BlockSpec((1, NH), lambda i: (0, 0)),
            pl.BlockSpec((NH, FE), lambda i: (0, 0)),
            pl.BlockSpec((1, FE), lambda i: (0, 0)),
        ],
        out_specs=pl.BlockSpec((R, FE), lambda i: (i, 0)),
        out_shape=jax.ShapeDtypeStruct((N_NODES, FE), jnp.float32),
    )(feats, p0, p1, wf, wa, b1, w2, b2)


def kernel(x, edge_index, edge_attr, We1, be1, We2, be2, Wn1, bn1, Wn2, bn2):
    feats = x[:, POS:]
    dst1 = edge_index[1]
    src1 = edge_index[0]

    a, b = _pre(feats, We1[:FE], We1[FE:2 * FE])
    eg = _gather(a, b, dst1, src1)
    eye4 = jnp.eye(4, dtype=jnp.float32)
    m2p = _edge(eg, edge_attr.reshape(EP, 4 * EA),
                jnp.kron(eye4, We1[2 * FE:]),
                jnp.tile(be1, 4).reshape(1, 4 * EH),
                jnp.kron(eye4, We2),
                jnp.tile(be2, 4).reshape(1, 4 * MD))
    parts = _scatter(m2p.reshape(N_EDGES, MD), dst1)
    feats_out = _node(feats, parts[0], parts[1], Wn1[:FE], Wn1[FE:],
                      bn1.reshape(1, NH), Wn2, bn2.reshape(1, FE))
    return jnp.concatenate([x[:, :POS], feats_out], axis=1)
